# baseline (speedup 1.0000x reference)
import functools

import jax
import jax.numpy as jnp
from jax import lax
from jax.experimental import pallas as pl
from jax.experimental.pallas import tpu as pltpu
from jax.experimental.pallas import tpu_sc as plsc

_N = 10000
_D = 128
_G = 64
_EPS = 1e-5
_NROWS = 10112
_RPT = _NROWS // 16
_EB = 128
_E = 320000
_NBLK = 160
_EPT = _EB * _NBLK
_EPAD = 16 * _EPT
_NBT = 16 * _NBLK

@functools.cache
def _mesh():
    return plsc.VectorSubcoreMesh(core_axis_name="c", subcore_axis_name="s")


def _sc_msgpass(hs, src2, dst2):
    @functools.partial(
        pl.kernel,
        out_type=jax.ShapeDtypeStruct((2, _NROWS, _D), jnp.float32),
        mesh=_mesh(),
        scratch_types=[
            pltpu.VMEM((_EB,), jnp.int32),
            pltpu.VMEM((_EB,), jnp.int32),
            pltpu.VMEM((_EB, _D), jnp.float32),
            pltpu.VMEM_SHARED((_NROWS, _D), jnp.float32),
        ],
    )
    def k(hs_hbm, src_hbm, dst_hbm, out_hbm, sx0, dx0, rows0, acc_sh):
        c = lax.axis_index("c")
        s = lax.axis_index("s")
        r0 = s * _RPT
        hs_c = hs_hbm.at[c]
        src_c = src_hbm.at[c]
        dst_c = dst_hbm.at[c]
        base = s * _EPT

        pltpu.sync_copy(hs_c.at[pl.ds(r0, _RPT)], acc_sh.at[pl.ds(r0, _RPT)])
        plsc.subcore_barrier()

        @pl.loop(0, _NBLK)
        def _(i):
            pltpu.sync_copy(src_c.at[pl.ds(base + i * _EB, _EB)], sx0)
            pltpu.sync_copy(dst_c.at[pl.ds(base + i * _EB, _EB)], dx0)
            pltpu.sync_copy(hs_c.at[sx0], rows0)
            pltpu.sync_copy(rows0, acc_sh.at[dx0], add=True)

        plsc.subcore_barrier()
        pltpu.sync_copy(acc_sh.at[pl.ds(r0, _RPT)],
                        out_hbm.at[c].at[pl.ds(r0, _RPT)])

    return k(hs, src2, dst2)


def _tc_pre(X, gamma, beta, W1, b1, W2, b2):
    def body(x_ref, g_ref, be_ref, w1_ref, b1_ref, w2_ref, b2_ref, o_ref):
        x = x_ref[0]
        mean = jnp.mean(x, axis=0, keepdims=True)
        xm = x - mean
        var = jnp.mean(xm * xm, axis=0, keepdims=True)
        xn = xm * lax.rsqrt(var + _EPS) * g_ref[...] + be_ref[...]
        y = jnp.maximum(jnp.dot(xn, w1_ref[...],
                                preferred_element_type=jnp.float32)
                        + b1_ref[...], 0.0)
        y = jnp.maximum(jnp.dot(y, w2_ref[...],
                                preferred_element_type=jnp.float32)
                        + b2_ref[...], 0.0)
        o_ref[0] = y

    full = lambda shape: pl.BlockSpec(shape, lambda b: (0,) * len(shape))
    return pl.pallas_call(
        body,
        grid=(2,),
        in_specs=[
            pl.BlockSpec((1, _N, _D), lambda b: (b, 0, 0)),
            full((1, _D)), full((1, _D)),
            full((_D, 2 * _D)), full((1, 2 * _D)),
            full((2 * _D, _D)), full((1, _D)),
        ],
        out_specs=pl.BlockSpec((1, _N, _D), lambda b: (b, 0, 0)),
        out_shape=jax.ShapeDtypeStruct((2, _N, _D), jnp.float32),
    )(X, gamma, beta, W1, b1, W2, b2)


def _dinv(deg_ref):
    return lax.rsqrt(deg_ref[0, :, 0:1])


def _tc_hs(Ypad, degraw, Wc):
    def body(y_ref, d_ref, w_ref, o_ref):
        h = jnp.dot(y_ref[0], w_ref[...], preferred_element_type=jnp.float32)
        o_ref[0] = h * _dinv(d_ref)

    return pl.pallas_call(
        body,
        grid=(2,),
        in_specs=[
            pl.BlockSpec((1, _NROWS, _D), lambda b: (b, 0, 0)),
            pl.BlockSpec((1, _NROWS, _D), lambda b: (b, 0, 0)),
            pl.BlockSpec((_D, _D), lambda b: (0, 0)),
        ],
        out_specs=pl.BlockSpec((1, _NROWS, _D), lambda b: (b, 0, 0)),
        out_shape=jax.ShapeDtypeStruct((2, _NROWS, _D), jnp.float32),
    )(Ypad, degraw, Wc)


def _tc_stage(agg, degraw, bc, Wnext):
    def body(a_ref, d_ref, b_ref, w_ref, o_ref):
        dinv = _dinv(d_ref)
        x = jnp.maximum(a_ref[0] * dinv + b_ref[...], 0.0)
        rows = lax.broadcasted_iota(jnp.int32, (_NROWS, 1), 0)
        x = jnp.where(rows < _N, x, 0.0)
        h = jnp.dot(x, w_ref[...], preferred_element_type=jnp.float32)
        o_ref[0] = h * dinv

    return pl.pallas_call(
        body,
        grid=(2,),
        in_specs=[
            pl.BlockSpec((1, _NROWS, _D), lambda b: (b, 0, 0)),
            pl.BlockSpec((1, _NROWS, _D), lambda b: (b, 0, 0)),
            pl.BlockSpec((1, _D), lambda b: (0, 0)),
            pl.BlockSpec((_D, _D), lambda b: (0, 0)),
        ],
        out_specs=pl.BlockSpec((1, _NROWS, _D), lambda b: (b, 0, 0)),
        out_shape=jax.ShapeDtypeStruct((2, _NROWS, _D), jnp.float32),
    )(agg, degraw, bc, Wnext)


def _tc_pool(agg, degraw, bc, batch2):
    def body(a_ref, d_ref, b_ref, bt_ref, o_ref):
        dinv = _dinv(d_ref)[: _N]
        x = jnp.maximum(a_ref[0, : _N, :] * dinv + b_ref[...], 0.0)
        gid = bt_ref[0]
        gids = lax.broadcasted_iota(jnp.int32, (1, _G), 1)
        oh = (gid == gids).astype(jnp.float32)
        s = lax.dot_general(oh, x, (((0,), (0,)), ((), ())),
                            preferred_element_type=jnp.float32)
        cnt = lax.dot_general(oh, jnp.ones((_N, _D), jnp.float32),
                              (((0,), (0,)), ((), ())),
                              preferred_element_type=jnp.float32)
        o_ref[0] = s / jnp.maximum(cnt, 1.0)

    return pl.pallas_call(
        body,
        grid=(2,),
        in_specs=[
            pl.BlockSpec((1, _NROWS, _D), lambda b: (b, 0, 0)),
            pl.BlockSpec((1, _NROWS, _D), lambda b: (b, 0, 0)),
            pl.BlockSpec((1, _D), lambda b: (0, 0)),
            pl.BlockSpec((1, _N, 1), lambda b: (b, 0, 0)),
        ],
        out_specs=pl.BlockSpec((1, _G, _D), lambda b: (b, 0, 0)),
        out_shape=jax.ShapeDtypeStruct((2, _G, _D), jnp.float32),
    )(agg, degraw, bc, batch2)


def _tc_head(h, W3, b3, W4, b4, W5, b5):
    def body(h_ref, w3_ref, b3_ref, w4_ref, b4_ref, w5_ref, b5_ref, o_ref):
        y = jnp.maximum(jnp.dot(h_ref[...], w3_ref[...],
                                preferred_element_type=jnp.float32)
                        + b3_ref[...], 0.0)
        y = jnp.maximum(jnp.dot(y, w4_ref[...],
                                preferred_element_type=jnp.float32)
                        + b4_ref[...], 0.0)
        o_ref[...] = jnp.dot(y, w5_ref[...],
                             preferred_element_type=jnp.float32) + b5_ref[...]

    return pl.pallas_call(
        body,
        out_shape=jax.ShapeDtypeStruct((_G, 1), jnp.float32),
    )(h, W3, b3, W4, b4, W5, b5)


def kernel(x0, x1, edge_index0, edge_index1, batch0, batch1, gamma, beta,
           W1, b1, W2, b2, Wc1, bc1, Wc2, bc2, Wc3, bc3, W3, b3, W4, b4,
           W5, b5):
    ei0 = edge_index0.astype(jnp.int32)
    ei1 = edge_index1.astype(jnp.int32)
    npad = _EPAD + 2 * _EB - _E
    pad = jnp.broadcast_to(
        _N + jnp.arange(npad, dtype=jnp.int32) % (_NROWS - _N), (2, npad))
    src2 = jnp.concatenate([jnp.stack([ei0[0], ei1[0]]), pad], axis=1)
    dst2 = jnp.concatenate([jnp.stack([ei0[1], ei1[1]]), pad], axis=1)
    X = jnp.stack([x0, x1])
    batch2 = jnp.stack([batch0, batch1]).astype(jnp.int32).reshape(2, _N, 1)
    ones128 = jnp.broadcast_to(
        (jnp.arange(_NROWS) < _N).astype(jnp.float32)[None, :, None],
        (2, _NROWS, _D))

    degraw = _sc_msgpass(ones128, src2, dst2)
    Y = _tc_pre(X, gamma.reshape(1, _D), beta.reshape(1, _D),
                W1, b1.reshape(1, 2 * _D), W2, b2.reshape(1, _D))
    Ypad = jnp.pad(Y, ((0, 0), (0, _NROWS - _N), (0, 0)))
    hs = _tc_hs(Ypad, degraw, Wc1)
    agg = _sc_msgpass(hs, src2, dst2)
    hs = _tc_stage(agg, degraw, bc1.reshape(1, _D), Wc2)
    agg = _sc_msgpass(hs, src2, dst2)
    hs = _tc_stage(agg, degraw, bc2.reshape(1, _D), Wc3)
    agg = _sc_msgpass(hs, src2, dst2)
    P = _tc_pool(agg, degraw, bc3.reshape(1, _D), batch2)
    h = jnp.concatenate([P[0], P[1]], axis=1)
    return _tc_head(h, W3, b3.reshape(1, 256), W4, b4.reshape(1, _D),
                    W5, b5.reshape(1, 1))


# speedup vs baseline: 1.8812x; 1.8812x over previous
import functools

import jax
import jax.numpy as jnp
from jax import lax
from jax.experimental import pallas as pl
from jax.experimental.pallas import tpu as pltpu
from jax.experimental.pallas import tpu_sc as plsc

_N = 10000
_D = 128
_G = 64
_EPS = 1e-5
_NROWS = 10112
_RPT = _NROWS // 16
_EB = 128
_E = 320000
_NBLK = 160
_EPT = _EB * _NBLK
_EPAD = 16 * _EPT
_NBT = 16 * _NBLK

@functools.cache
def _mesh():
    return plsc.VectorSubcoreMesh(core_axis_name="c", subcore_axis_name="s")


def _sc_msgpass(hs, src2, dst2):
    @functools.partial(
        pl.kernel,
        out_type=jax.ShapeDtypeStruct((2, _NROWS, _D), jnp.float32),
        mesh=_mesh(),
        scratch_types=[
            pltpu.VMEM((_EB,), jnp.int32),
            pltpu.VMEM((_EB,), jnp.int32),
            pltpu.VMEM((_EB,), jnp.int32),
            pltpu.VMEM((_EB,), jnp.int32),
            pltpu.VMEM((_EB,), jnp.int32),
            pltpu.VMEM((_EB,), jnp.int32),
            pltpu.VMEM((_EB,), jnp.int32),
            pltpu.VMEM((_EB,), jnp.int32),
            pltpu.VMEM((_EB, _D), jnp.float32),
            pltpu.VMEM((_EB, _D), jnp.float32),
            pltpu.VMEM_SHARED((_NROWS, _D), jnp.float32),
            pltpu.SemaphoreType.DMA,
            pltpu.SemaphoreType.DMA,
            pltpu.SemaphoreType.DMA,
            pltpu.SemaphoreType.DMA,
            pltpu.SemaphoreType.DMA,
            pltpu.SemaphoreType.DMA,
            pltpu.SemaphoreType.DMA,
            pltpu.SemaphoreType.DMA,
            pltpu.SemaphoreType.DMA,
            pltpu.SemaphoreType.DMA,
            pltpu.SemaphoreType.DMA,
            pltpu.SemaphoreType.DMA,
        ],
    )
    def k(hs_hbm, src_hbm, dst_hbm, out_hbm,
          sx0, sx1, sx2, sx3, dx0, dx1, dx2, dx3, rows0, rows1, acc_sh,
          si0, si1, si2, si3, di0, di1, di2, di3, g0, g1, s0, s1):
        c = lax.axis_index("c")
        s = lax.axis_index("s")
        r0 = s * _RPT
        hs_c = hs_hbm.at[c]
        src_c = src_hbm.at[c]
        dst_c = dst_hbm.at[c]
        base = s * _EPT
        sidx = (sx0, sx1, sx2, sx3)
        didx = (dx0, dx1, dx2, dx3)
        sisem = (si0, si1, si2, si3)
        disem = (di0, di1, di2, di3)
        rows = (rows0, rows1)
        gsem = (g0, g1)
        ssem = (s0, s1)

        def sdma(i, q):
            return pltpu.make_async_copy(
                src_c.at[pl.ds(base + i * _EB, _EB)], sidx[q], sisem[q])

        def ddma(i, q):
            return pltpu.make_async_copy(
                dst_c.at[pl.ds(base + i * _EB, _EB)], didx[q], disem[q])

        def gat(q, b):
            return pltpu.make_async_copy(hs_c.at[sidx[q]], rows[b], gsem[b])

        def scat(q, b):
            return pltpu.make_async_copy(rows[b], acc_sh.at[didx[q]], ssem[b])

        pltpu.sync_copy(hs_c.at[pl.ds(r0, _RPT)], acc_sh.at[pl.ds(r0, _RPT)])
        plsc.subcore_barrier()

        for q in range(3):
            sdma(q, q).start()
            ddma(q, q).start()
        sdma(0, 0).wait()
        gat(0, 0).start()
        gat(0, 0).wait()
        sdma(3, 3).start()
        ddma(3, 3).start()
        sdma(1, 1).wait()
        ddma(1, 1).wait()
        gat(1, 1).start()
        ddma(0, 0).wait()
        scat(0, 0).start(add=True)

        @pl.loop(1, _NBLK - 3, step=4)
        def _(j):
            for t in range(4):
                i = j + t
                b = (t + 1) & 1
                q = (t + 1) & 3
                qp = t & 3
                qn = (t + 2) & 3
                gat(q, b).wait()
                scat(qp, b ^ 1).wait()
                sdma(i + 3, qp).start()
                ddma(i + 3, qp).start()
                sdma(i + 1, qn).wait()
                ddma(i + 1, qn).wait()
                gat(qn, b ^ 1).start()
                scat(q, b).start(add=True)

        gat(1, 1).wait()
        scat(0, 0).wait()
        sdma(_NBLK - 2, 2).wait()
        ddma(_NBLK - 2, 2).wait()
        gat(2, 0).start()
        scat(1, 1).start(add=True)

        gat(2, 0).wait()
        scat(1, 1).wait()
        sdma(_NBLK - 1, 3).wait()
        ddma(_NBLK - 1, 3).wait()
        gat(3, 1).start()
        scat(2, 0).start(add=True)

        gat(3, 1).wait()
        scat(2, 0).wait()
        scat(3, 1).start(add=True)
        scat(3, 1).wait()

        plsc.subcore_barrier()
        pltpu.sync_copy(acc_sh.at[pl.ds(r0, _RPT)],
                        out_hbm.at[c].at[pl.ds(r0, _RPT)])

    return k(hs, src2, dst2)


def _tc_pre(X, gamma, beta, W1, b1, W2, b2):
    def body(x_ref, g_ref, be_ref, w1_ref, b1_ref, w2_ref, b2_ref, o_ref):
        x = x_ref[0]
        mean = jnp.mean(x, axis=0, keepdims=True)
        xm = x - mean
        var = jnp.mean(xm * xm, axis=0, keepdims=True)
        xn = xm * lax.rsqrt(var + _EPS) * g_ref[...] + be_ref[...]
        y = jnp.maximum(jnp.dot(xn, w1_ref[...],
                                preferred_element_type=jnp.float32)
                        + b1_ref[...], 0.0)
        y = jnp.maximum(jnp.dot(y, w2_ref[...],
                                preferred_element_type=jnp.float32)
                        + b2_ref[...], 0.0)
        o_ref[0] = y

    full = lambda shape: pl.BlockSpec(shape, lambda b: (0,) * len(shape))
    return pl.pallas_call(
        body,
        grid=(2,),
        in_specs=[
            pl.BlockSpec((1, _N, _D), lambda b: (b, 0, 0)),
            full((1, _D)), full((1, _D)),
            full((_D, 2 * _D)), full((1, 2 * _D)),
            full((2 * _D, _D)), full((1, _D)),
        ],
        out_specs=pl.BlockSpec((1, _N, _D), lambda b: (b, 0, 0)),
        out_shape=jax.ShapeDtypeStruct((2, _N, _D), jnp.float32),
    )(X, gamma, beta, W1, b1, W2, b2)


def _dinv(deg_ref):
    return lax.rsqrt(deg_ref[0, :, 0:1])


def _tc_hs(Ypad, degraw, Wc):
    def body(y_ref, d_ref, w_ref, o_ref):
        h = jnp.dot(y_ref[0], w_ref[...], preferred_element_type=jnp.float32)
        o_ref[0] = h * _dinv(d_ref)

    return pl.pallas_call(
        body,
        grid=(2,),
        in_specs=[
            pl.BlockSpec((1, _NROWS, _D), lambda b: (b, 0, 0)),
            pl.BlockSpec((1, _NROWS, _D), lambda b: (b, 0, 0)),
            pl.BlockSpec((_D, _D), lambda b: (0, 0)),
        ],
        out_specs=pl.BlockSpec((1, _NROWS, _D), lambda b: (b, 0, 0)),
        out_shape=jax.ShapeDtypeStruct((2, _NROWS, _D), jnp.float32),
    )(Ypad, degraw, Wc)


def _tc_stage(agg, degraw, bc, Wnext):
    def body(a_ref, d_ref, b_ref, w_ref, o_ref):
        dinv = _dinv(d_ref)
        x = jnp.maximum(a_ref[0] * dinv + b_ref[...], 0.0)
        rows = lax.broadcasted_iota(jnp.int32, (_NROWS, 1), 0)
        x = jnp.where(rows < _N, x, 0.0)
        h = jnp.dot(x, w_ref[...], preferred_element_type=jnp.float32)
        o_ref[0] = h * dinv

    return pl.pallas_call(
        body,
        grid=(2,),
        in_specs=[
            pl.BlockSpec((1, _NROWS, _D), lambda b: (b, 0, 0)),
            pl.BlockSpec((1, _NROWS, _D), lambda b: (b, 0, 0)),
            pl.BlockSpec((1, _D), lambda b: (0, 0)),
            pl.BlockSpec((_D, _D), lambda b: (0, 0)),
        ],
        out_specs=pl.BlockSpec((1, _NROWS, _D), lambda b: (b, 0, 0)),
        out_shape=jax.ShapeDtypeStruct((2, _NROWS, _D), jnp.float32),
    )(agg, degraw, bc, Wnext)


def _tc_pool(agg, degraw, bc, batch2):
    def body(a_ref, d_ref, b_ref, bt_ref, o_ref):
        dinv = _dinv(d_ref)[: _N]
        x = jnp.maximum(a_ref[0, : _N, :] * dinv + b_ref[...], 0.0)
        gid = bt_ref[0]
        gids = lax.broadcasted_iota(jnp.int32, (1, _G), 1)
        oh = (gid == gids).astype(jnp.float32)
        s = lax.dot_general(oh, x, (((0,), (0,)), ((), ())),
                            preferred_element_type=jnp.float32)
        cnt = lax.dot_general(oh, jnp.ones((_N, _D), jnp.float32),
                              (((0,), (0,)), ((), ())),
                              preferred_element_type=jnp.float32)
        o_ref[0] = s / jnp.maximum(cnt, 1.0)

    return pl.pallas_call(
        body,
        grid=(2,),
        in_specs=[
            pl.BlockSpec((1, _NROWS, _D), lambda b: (b, 0, 0)),
            pl.BlockSpec((1, _NROWS, _D), lambda b: (b, 0, 0)),
            pl.BlockSpec((1, _D), lambda b: (0, 0)),
            pl.BlockSpec((1, _N, 1), lambda b: (b, 0, 0)),
        ],
        out_specs=pl.BlockSpec((1, _G, _D), lambda b: (b, 0, 0)),
        out_shape=jax.ShapeDtypeStruct((2, _G, _D), jnp.float32),
    )(agg, degraw, bc, batch2)


def _tc_head(h, W3, b3, W4, b4, W5, b5):
    def body(h_ref, w3_ref, b3_ref, w4_ref, b4_ref, w5_ref, b5_ref, o_ref):
        y = jnp.maximum(jnp.dot(h_ref[...], w3_ref[...],
                                preferred_element_type=jnp.float32)
                        + b3_ref[...], 0.0)
        y = jnp.maximum(jnp.dot(y, w4_ref[...],
                                preferred_element_type=jnp.float32)
                        + b4_ref[...], 0.0)
        o_ref[...] = jnp.dot(y, w5_ref[...],
                             preferred_element_type=jnp.float32) + b5_ref[...]

    return pl.pallas_call(
        body,
        out_shape=jax.ShapeDtypeStruct((_G, 1), jnp.float32),
    )(h, W3, b3, W4, b4, W5, b5)


def kernel(x0, x1, edge_index0, edge_index1, batch0, batch1, gamma, beta,
           W1, b1, W2, b2, Wc1, bc1, Wc2, bc2, Wc3, bc3, W3, b3, W4, b4,
           W5, b5):
    ei0 = edge_index0.astype(jnp.int32)
    ei1 = edge_index1.astype(jnp.int32)
    npad = _EPAD + 2 * _EB - _E
    pad = jnp.broadcast_to(
        _N + jnp.arange(npad, dtype=jnp.int32) % (_NROWS - _N), (2, npad))
    src2 = jnp.concatenate([jnp.stack([ei0[0], ei1[0]]), pad], axis=1)
    dst2 = jnp.concatenate([jnp.stack([ei0[1], ei1[1]]), pad], axis=1)
    X = jnp.stack([x0, x1])
    batch2 = jnp.stack([batch0, batch1]).astype(jnp.int32).reshape(2, _N, 1)
    ones128 = jnp.broadcast_to(
        (jnp.arange(_NROWS) < _N).astype(jnp.float32)[None, :, None],
        (2, _NROWS, _D))

    degraw = _sc_msgpass(ones128, src2, dst2)
    Y = _tc_pre(X, gamma.reshape(1, _D), beta.reshape(1, _D),
                W1, b1.reshape(1, 2 * _D), W2, b2.reshape(1, _D))
    Ypad = jnp.pad(Y, ((0, 0), (0, _NROWS - _N), (0, 0)))
    hs = _tc_hs(Ypad, degraw, Wc1)
    agg = _sc_msgpass(hs, src2, dst2)
    hs = _tc_stage(agg, degraw, bc1.reshape(1, _D), Wc2)
    agg = _sc_msgpass(hs, src2, dst2)
    hs = _tc_stage(agg, degraw, bc2.reshape(1, _D), Wc3)
    agg = _sc_msgpass(hs, src2, dst2)
    P = _tc_pool(agg, degraw, bc3.reshape(1, _D), batch2)
    h = jnp.concatenate([P[0], P[1]], axis=1)
    return _tc_head(h, W3, b3.reshape(1, 256), W4, b4.reshape(1, _D),
                    W5, b5.reshape(1, 1))
